# Initial kernel scaffold; baseline (speedup 1.0000x reference)
import jax
import jax.numpy as jnp
from jax.experimental import pallas as pl


def kernel(ts, embedding):
    raise NotImplementedError("write your pallas kernel here")


# speedup vs baseline: 1.4357x; 1.4357x over previous
import functools

import jax
import jax.numpy as jnp
from jax import lax
from jax.experimental import pallas as pl
from jax.experimental.pallas import tpu as pltpu
from jax.experimental.pallas import tpu_sc as plsc

SEQ = 200
EXP = 32
ROWS_TC = 32
NW = 32
IDX_MINOR = 128
SB_CHUNKS = 32
TBL_ROWS = 2 * SEQ


def _rank_body(ts_ref, g_ref):
    ts = ts_ref[...]
    acc = jnp.zeros(ts.shape, jnp.float32)
    lane = lax.broadcasted_iota(jnp.int32, ts.shape, 1)
    cur = ts
    for s in range(1, SEQ):
        cur = jnp.concatenate([cur[:, 1:], cur[:, :1]], axis=1)
        lt = cur < ts
        tie = (cur == ts) & (lane >= (SEQ - s))
        acc = acc + jnp.where(lt | tie, 1.0, 0.0)
    base = pl.program_id(0) * ROWS_TC
    row = base + lax.broadcasted_iota(jnp.int32, ts.shape, 0)
    g_ref[...] = acc.astype(jnp.int32) + row * SEQ


def _ranks_tc(ts):
    batch = ts.shape[0]
    return pl.pallas_call(
        _rank_body,
        grid=(batch // ROWS_TC,),
        in_specs=[pl.BlockSpec((ROWS_TC, SEQ), lambda i: (i, 0))],
        out_specs=pl.BlockSpec((ROWS_TC, SEQ), lambda i: (i, 0)),
        out_shape=jax.ShapeDtypeStruct((batch, SEQ), jnp.int32),
    )(ts)


def _scatter_sc(g2, embedding, batch):
    flat = batch * SEQ
    nchunks_w = flat // (NW * IDX_MINOR)
    nsb = nchunks_w // SB_CHUNKS
    mesh = plsc.VectorSubcoreMesh(core_axis_name="c", subcore_axis_name="s")

    @functools.partial(
        pl.kernel,
        mesh=mesh,
        out_type=jax.ShapeDtypeStruct((flat, EXP), jnp.float32),
        scratch_types=[
            pltpu.VMEM((SB_CHUNKS, IDX_MINOR), jnp.int32),
            pltpu.VMEM((TBL_ROWS, EXP), jnp.float32),
            pltpu.SemaphoreType.DMA,
        ],
        compiler_params=pltpu.CompilerParams(use_tc_tiling_on_sc=False),
    )
    def k(g_hbm, emb_hbm, out_hbm, idx_v, tbl_v, sem):
        wid = lax.axis_index("s") * 2 + lax.axis_index("c")
        for i in range(TBL_ROWS // SEQ):
            pltpu.sync_copy(emb_hbm, tbl_v.at[pl.ds(i * SEQ, SEQ)])

        def chunk_copy(cbase, c):
            off = pl.multiple_of(lax.rem((cbase + c) * IDX_MINOR, SEQ), 8)
            return pltpu.make_async_copy(
                tbl_v.at[pl.ds(off, IDX_MINOR)],
                out_hbm.at[idx_v.at[c]],
                sem,
            )

        def body(sb, carry):
            cbase = pl.multiple_of(wid * nchunks_w + sb * SB_CHUNKS, 8)
            pltpu.sync_copy(g_hbm.at[pl.ds(cbase, SB_CHUNKS)], idx_v)
            lax.fori_loop(0, SB_CHUNKS, lambda c, x: (chunk_copy(cbase, c).start(), x)[1], 0)
            lax.fori_loop(0, SB_CHUNKS, lambda c, x: (chunk_copy(cbase, c).wait(), x)[1], 0)
            return carry

        lax.fori_loop(0, nsb, body, 0)

    return k(g2, embedding)


def kernel(ts, embedding):
    batch = ts.shape[0]
    g = _ranks_tc(ts)
    g2 = g.reshape(batch * SEQ // IDX_MINOR, IDX_MINOR)
    out = _scatter_sc(g2, embedding, batch)
    return out.reshape(batch, SEQ, EXP)


# speedup vs baseline: 3.6813x; 2.5642x over previous
import functools

import jax
import jax.numpy as jnp
from jax import lax
from jax.experimental import pallas as pl
from jax.experimental.pallas import tpu as pltpu
from jax.experimental.pallas import tpu_sc as plsc

SEQ = 200
EXP = 32
BG = 128
NW = 32
LANES = 16


def _rank_body(ts_ref, g_ref):
    tst = ts_ref[...].T
    jio = lax.broadcasted_iota(jnp.int32, (SEQ, BG), 0)
    acc = jnp.zeros((SEQ, BG), jnp.float32)
    for k in range(SEQ):
        bk = jnp.broadcast_to(tst[k:k + 1, :], (SEQ, BG))
        lt = bk < tst
        tie = (bk == tst) & (jio > k)
        acc = acc + jnp.where(lt | tie, 1.0, 0.0)
    b = pl.program_id(0) * BG + lax.broadcasted_iota(jnp.int32, (SEQ, BG), 1)
    g_ref[0] = acc.astype(jnp.int32) + b * SEQ


def _ranks_tc(ts):
    batch = ts.shape[0]
    ngrp = batch // BG
    return pl.pallas_call(
        _rank_body,
        grid=(ngrp,),
        in_specs=[pl.BlockSpec((BG, SEQ), lambda i: (i, 0))],
        out_specs=pl.BlockSpec((1, SEQ, BG), lambda i: (i, 0, 0)),
        out_shape=jax.ShapeDtypeStruct((ngrp, SEQ, BG), jnp.int32),
    )(ts)


def _scatter_sc(g2, embedding, batch):
    flat = batch * SEQ
    ngrp = batch // BG
    gpw = ngrp // NW
    mesh = plsc.VectorSubcoreMesh(core_axis_name="c", subcore_axis_name="s")

    @functools.partial(
        pl.kernel,
        mesh=mesh,
        out_type=jax.ShapeDtypeStruct((flat, EXP), jnp.float32),
        scratch_types=[
            pltpu.VMEM((gpw * SEQ, BG), jnp.int32),
            pltpu.VMEM((BG, EXP), jnp.float32),
            pltpu.VMEM((BG,), jnp.int32),
            pltpu.SemaphoreType.DMA,
            pltpu.SemaphoreType.DMA,
        ],
        compiler_params=pltpu.CompilerParams(use_tc_tiling_on_sc=False),
    )
    def k(g_hbm, emb_hbm, out_hbm, idx_v, repl_v, jrep_v, sem_g, sem_s):
        wid = lax.axis_index("s") * 2 + lax.axis_index("c")
        for t in range(gpw):
            src = pl.multiple_of((wid * gpw + t) * SEQ, 8)
            pltpu.sync_copy(g_hbm.at[pl.ds(src, SEQ)],
                            idx_v.at[pl.ds(t * SEQ, SEQ)])

        def body(j, carry):
            for v in range(BG // LANES):
                jrep_v[pl.ds(v * LANES, LANES)] = jnp.full((LANES,), j,
                                                           jnp.int32)
            pltpu.async_copy(emb_hbm.at[jrep_v], repl_v, sem_g).wait()

            def chunk(t):
                return pltpu.make_async_copy(
                    repl_v, out_hbm.at[idx_v.at[t * SEQ + j]], sem_s)

            for t in range(gpw):
                chunk(t).start()
            for t in range(gpw):
                chunk(t).wait()
            return carry

        lax.fori_loop(0, SEQ, body, 0)

    return k(g2, embedding)


def kernel(ts, embedding):
    batch = ts.shape[0]
    g = _ranks_tc(ts)
    g2 = g.reshape(batch // BG * SEQ, BG)
    out = _scatter_sc(g2, embedding, batch)
    return out.reshape(batch, SEQ, EXP)


# speedup vs baseline: 3.8208x; 1.0379x over previous
import functools

import jax
import jax.numpy as jnp
from jax import lax
from jax.experimental import pallas as pl
from jax.experimental.pallas import tpu as pltpu
from jax.experimental.pallas import tpu_sc as plsc

SEQ = 200
EXP = 32
BG = 128
NW = 32
LANES = 16


def _rank_body(ts_ref, g_ref):
    tst = ts_ref[...].T
    jio = lax.broadcasted_iota(jnp.int32, (SEQ, BG), 0)
    acc = jnp.zeros((SEQ, BG), jnp.float32)
    for k in range(SEQ):
        bk = jnp.broadcast_to(tst[k:k + 1, :], (SEQ, BG))
        lt = bk < tst
        tie = (bk == tst) & (jio > k)
        acc = acc + jnp.where(lt | tie, 1.0, 0.0)
    b = pl.program_id(0) * BG + lax.broadcasted_iota(jnp.int32, (SEQ, BG), 1)
    g_ref[0] = acc.astype(jnp.int32) + b * SEQ


def _ranks_tc(ts):
    batch = ts.shape[0]
    ngrp = batch // BG
    return pl.pallas_call(
        _rank_body,
        grid=(ngrp,),
        in_specs=[pl.BlockSpec((BG, SEQ), lambda i: (i, 0))],
        out_specs=pl.BlockSpec((1, SEQ, BG), lambda i: (i, 0, 0)),
        out_shape=jax.ShapeDtypeStruct((ngrp, SEQ, BG), jnp.int32),
    )(ts)


def _scatter_sc(g2, embedding, batch):
    flat = batch * SEQ
    ngrp = batch // BG
    gpw = ngrp // NW
    mesh = plsc.VectorSubcoreMesh(core_axis_name="c", subcore_axis_name="s")

    @functools.partial(
        pl.kernel,
        mesh=mesh,
        out_type=jax.ShapeDtypeStruct((flat, EXP), jnp.float32),
        scratch_types=[
            pltpu.VMEM((gpw * SEQ, BG), jnp.int32),
            pltpu.VMEM((2, BG, EXP), jnp.float32),
            pltpu.VMEM((2, BG), jnp.int32),
            pltpu.SemaphoreType.DMA,
            pltpu.SemaphoreType.DMA,
        ],
        compiler_params=pltpu.CompilerParams(use_tc_tiling_on_sc=False),
    )
    def k(g_hbm, emb_hbm, out_hbm, idx_v, repl_v, jrep_v, sem_g, sem_s):
        wid = lax.axis_index("s") * 2 + lax.axis_index("c")
        for t in range(gpw):
            src = pl.multiple_of((wid * gpw + t) * SEQ, 8)
            pltpu.sync_copy(g_hbm.at[pl.ds(src, SEQ)],
                            idx_v.at[pl.ds(t * SEQ, SEQ)])

        def set_jrep(p, j):
            for v in range(BG // LANES):
                jrep_v[p, pl.ds(v * LANES, LANES)] = jnp.full(
                    (LANES,), j, jnp.int32)

        def gather(p):
            return pltpu.make_async_copy(
                emb_hbm.at[jrep_v.at[p]], repl_v.at[p], sem_g)

        def scat(p, t, j):
            return pltpu.make_async_copy(
                repl_v.at[p], out_hbm.at[idx_v.at[t * SEQ + j]], sem_s)

        set_jrep(0, 0)
        gather(0).start()

        def body(j, carry):
            p = lax.rem(j, 2)
            gather(p).wait()

            @pl.when(j >= 1)
            def _():
                for t in range(gpw):
                    scat(1 - p, t, j - 1).wait()

            @pl.when(j + 1 < SEQ)
            def _():
                set_jrep(1 - p, j + 1)
                gather(1 - p).start()

            for t in range(gpw):
                scat(p, t, j).start()
            return carry

        lax.fori_loop(0, SEQ, body, 0)
        for t in range(gpw):
            scat((SEQ - 1) % 2, t, SEQ - 1).wait()

    return k(g2, embedding)


def kernel(ts, embedding):
    batch = ts.shape[0]
    g = _ranks_tc(ts)
    g2 = g.reshape(batch // BG * SEQ, BG)
    out = _scatter_sc(g2, embedding, batch)
    return out.reshape(batch, SEQ, EXP)


# speedup vs baseline: 6.2073x; 1.6246x over previous
import functools

import jax
import jax.numpy as jnp
from jax import lax
from jax.experimental import pallas as pl
from jax.experimental.pallas import tpu as pltpu
from jax.experimental.pallas import tpu_sc as plsc

SEQ = 200
EXP = 32
BG = 128
NW = 32
NCH = SEQ
BLK = NCH * BG
TBL_ROWS = 2 * SEQ
INFLIGHT = 8


def _rank_body(ts_ref, g_ref):
    tst = ts_ref[...].T
    jio = lax.broadcasted_iota(jnp.int32, (SEQ, BG), 0)
    acc = jnp.zeros((SEQ, BG), jnp.float32)
    for k in range(SEQ):
        bk = jnp.broadcast_to(tst[k:k + 1, :], (SEQ, BG))
        lt = bk < tst
        tie = (bk == tst) & (jio > k)
        acc = acc + jnp.where(lt | tie, 1.0, 0.0)
    b = pl.program_id(0) * BG + lax.broadcasted_iota(jnp.int32, (BG, SEQ), 0)
    g_ref[...] = acc.T.astype(jnp.int32) + b * SEQ


def _ranks_tc(ts):
    batch = ts.shape[0]
    ngrp = batch // BG
    return pl.pallas_call(
        _rank_body,
        grid=(ngrp,),
        in_specs=[pl.BlockSpec((BG, SEQ), lambda i: (i, 0))],
        out_specs=pl.BlockSpec((BG, SEQ), lambda i: (i, 0)),
        out_shape=jax.ShapeDtypeStruct((batch, SEQ), jnp.int32),
    )(ts)


def _scatter_sc(g2d, embedding, batch):
    flat = batch * SEQ
    ngrp = batch // BG
    gpw = ngrp // NW
    mesh = plsc.VectorSubcoreMesh(core_axis_name="c", subcore_axis_name="s")

    @functools.partial(
        pl.kernel,
        mesh=mesh,
        out_type=jax.ShapeDtypeStruct((flat, EXP), jnp.float32),
        scratch_types=[
            pltpu.VMEM((NCH, BG), jnp.int32),
            pltpu.VMEM((NCH, BG), jnp.int32),
            pltpu.VMEM((TBL_ROWS, EXP), jnp.float32),
            pltpu.SemaphoreType.DMA,
            pltpu.SemaphoreType.DMA,
        ],
        compiler_params=pltpu.CompilerParams(use_tc_tiling_on_sc=False),
    )
    def k(g_hbm, emb_hbm, out_hbm, ids0_v, ids1_v, tbl_v, sem, lsem):
        wid = lax.axis_index("s") * 2 + lax.axis_index("c")
        for i in range(TBL_ROWS // SEQ):
            pltpu.sync_copy(emb_hbm, tbl_v.at[pl.ds(i * SEQ, SEQ)])

        slots = (ids0_v, ids1_v)

        def load(t, slot):
            base = (wid * gpw + t) * NCH
            return pltpu.make_async_copy(
                g_hbm.at[pl.ds(base, NCH)], slots[slot], lsem)

        def chunk_copy(ids_v, c):
            off = pl.multiple_of(lax.rem(c * BG, SEQ), 8)
            return pltpu.make_async_copy(
                tbl_v.at[pl.ds(off, BG)], out_hbm.at[ids_v.at[c]], sem)

        load(0, 0).start()
        for t in range(gpw):
            ids_v = slots[t & 1]
            load(t, t & 1).wait()
            if t + 1 < gpw:
                load(t + 1, (t + 1) & 1).start()

            def scat_step(c, carry):
                chunk_copy(ids_v, c).start()

                @pl.when(c >= INFLIGHT)
                def _():
                    chunk_copy(ids_v, c - INFLIGHT).wait()
                return carry

            lax.fori_loop(0, NCH, scat_step, 0)
            for c in range(NCH - INFLIGHT, NCH):
                chunk_copy(ids_v, c).wait()

    return k(g2d, embedding)


def kernel(ts, embedding):
    batch = ts.shape[0]
    g = _ranks_tc(ts)
    g2d = g.reshape(batch * SEQ // BG, BG)
    out = _scatter_sc(g2d, embedding, batch)
    return out.reshape(batch, SEQ, EXP)


# speedup vs baseline: 6.2099x; 1.0004x over previous
import functools

import jax
import jax.numpy as jnp
from jax import lax
from jax.experimental import pallas as pl
from jax.experimental.pallas import tpu as pltpu
from jax.experimental.pallas import tpu_sc as plsc

SEQ = 200
EXP = 32
BG = 128
NW = 32
INFLIGHT = 8


def _rank_body(ts_ref, g_ref):
    tst = ts_ref[...].T
    jio = lax.broadcasted_iota(jnp.int32, (SEQ, BG), 0)
    acc = jnp.zeros((SEQ, BG), jnp.float32)
    for k in range(SEQ):
        bk = jnp.broadcast_to(tst[k:k + 1, :], (SEQ, BG))
        lt = bk < tst
        tie = (bk == tst) & (jio > k)
        acc = acc + jnp.where(lt | tie, 1.0, 0.0)
    b = pl.program_id(0) * BG + lax.broadcasted_iota(jnp.int32, (BG, SEQ), 0)
    g_ref[...] = acc.T.astype(jnp.int32) + b * SEQ


def _ranks_tc(ts):
    batch = ts.shape[0]
    ngrp = batch // BG
    return pl.pallas_call(
        _rank_body,
        grid=(ngrp,),
        in_specs=[pl.BlockSpec((BG, SEQ), lambda i: (i, 0))],
        out_specs=pl.BlockSpec((BG, SEQ), lambda i: (i, 0)),
        out_shape=jax.ShapeDtypeStruct((batch, SEQ), jnp.int32),
    )(ts)


def _scatter_sc(g, embedding, batch):
    flat = batch * SEQ
    ngrp = batch // BG
    gpw = ngrp // NW
    mesh = plsc.VectorSubcoreMesh(core_axis_name="c", subcore_axis_name="s")

    @functools.partial(
        pl.kernel,
        mesh=mesh,
        out_type=jax.ShapeDtypeStruct((flat, EXP), jnp.float32),
        scratch_types=[
            pltpu.VMEM((BG, SEQ), jnp.int32),
            pltpu.VMEM((BG, SEQ), jnp.int32),
            pltpu.VMEM((SEQ, EXP), jnp.float32),
            pltpu.SemaphoreType.DMA,
            pltpu.SemaphoreType.DMA,
        ],
        compiler_params=pltpu.CompilerParams(use_tc_tiling_on_sc=False),
    )
    def k(g_hbm, emb_hbm, out_hbm, ids0_v, ids1_v, tbl_v, sem, lsem):
        wid = lax.axis_index("s") * 2 + lax.axis_index("c")
        pltpu.sync_copy(emb_hbm, tbl_v)

        slots = (ids0_v, ids1_v)

        def load(t, slot):
            base = (wid * gpw + t) * BG
            return pltpu.make_async_copy(
                g_hbm.at[pl.ds(base, BG)], slots[slot], lsem)

        def chunk_copy(ids_v, c):
            return pltpu.make_async_copy(
                tbl_v, out_hbm.at[ids_v.at[c]], sem)

        load(0, 0).start()
        for t in range(gpw):
            ids_v = slots[t & 1]
            load(t, t & 1).wait()
            if t + 1 < gpw:
                load(t + 1, (t + 1) & 1).start()

            def scat_step(c, carry):
                chunk_copy(ids_v, c).start()

                @pl.when(c >= INFLIGHT)
                def _():
                    chunk_copy(ids_v, c - INFLIGHT).wait()
                return carry

            lax.fori_loop(0, BG, scat_step, 0)
            for c in range(BG - INFLIGHT, BG):
                chunk_copy(ids_v, c).wait()

    return k(g, embedding)


def kernel(ts, embedding):
    batch = ts.shape[0]
    g = _ranks_tc(ts)
    out = _scatter_sc(g, embedding, batch)
    return out.reshape(batch, SEQ, EXP)


# speedup vs baseline: 6.2150x; 1.0008x over previous
import functools

import jax
import jax.numpy as jnp
from jax import lax
from jax.experimental import pallas as pl
from jax.experimental.pallas import tpu as pltpu
from jax.experimental.pallas import tpu_sc as plsc

SEQ = 200
EXP = 32
BG = 128
NW = 32
INFLIGHT = 8


def _rank_body(ts_ref, g_ref):
    tst = ts_ref[...].T
    jio = lax.broadcasted_iota(jnp.int32, (SEQ, BG), 0)
    acc = jnp.zeros((SEQ, BG), jnp.float32)
    for k in range(SEQ):
        bk = jnp.broadcast_to(tst[k:k + 1, :], (SEQ, BG))
        lt = bk < tst
        tie = (bk == tst) & (jio > k)
        acc = acc + jnp.where(lt | tie, 1.0, 0.0)
    b = pl.program_id(0) * BG + lax.broadcasted_iota(jnp.int32, (BG, SEQ), 0)
    g_ref[...] = acc.T.astype(jnp.int32) + b * SEQ


def _ranks_tc(ts):
    batch = ts.shape[0]
    ngrp = batch // BG
    return pl.pallas_call(
        _rank_body,
        grid=(ngrp,),
        in_specs=[pl.BlockSpec((BG, SEQ), lambda i: (i, 0))],
        out_specs=pl.BlockSpec((BG, SEQ), lambda i: (i, 0)),
        out_shape=jax.ShapeDtypeStruct((batch, SEQ), jnp.int32),
    )(ts)


def _scatter_sc(g, embedding, batch):
    flat = batch * SEQ
    ngrp = batch // BG
    gpw = ngrp // NW
    mesh = plsc.VectorSubcoreMesh(core_axis_name="c", subcore_axis_name="s")

    @functools.partial(
        pl.kernel,
        mesh=mesh,
        out_type=jax.ShapeDtypeStruct((flat, EXP), jnp.float32),
        scratch_types=[
            pltpu.VMEM((BG * SEQ,), jnp.int32),
            pltpu.VMEM((BG * SEQ,), jnp.int32),
            pltpu.VMEM((SEQ, EXP), jnp.float32),
            pltpu.SemaphoreType.DMA,
            pltpu.SemaphoreType.DMA,
        ],
        compiler_params=pltpu.CompilerParams(use_tc_tiling_on_sc=False),
    )
    def k(g_hbm, emb_hbm, out_hbm, ids0_v, ids1_v, tbl_v, sem, lsem):
        wid = lax.axis_index("s") * 2 + lax.axis_index("c")
        pltpu.sync_copy(emb_hbm, tbl_v)

        slots = (ids0_v, ids1_v)

        def load(t, slot):
            base = (wid * gpw + t) * BG * SEQ
            return pltpu.make_async_copy(
                g_hbm.at[pl.ds(base, BG * SEQ)], slots[slot], lsem)

        def chunk_copy(ids_v, c):
            off = pl.multiple_of(c * SEQ, 8)
            return pltpu.make_async_copy(
                tbl_v, out_hbm.at[ids_v.at[pl.ds(off, SEQ)]], sem)

        load(0, 0).start()
        for t in range(gpw):
            ids_v = slots[t & 1]
            load(t, t & 1).wait()
            if t + 1 < gpw:
                load(t + 1, (t + 1) & 1).start()

            def scat_step(c, carry):
                chunk_copy(ids_v, c).start()

                @pl.when(c >= INFLIGHT)
                def _():
                    chunk_copy(ids_v, c - INFLIGHT).wait()
                return carry

            lax.fori_loop(0, BG, scat_step, 0)
            for c in range(BG - INFLIGHT, BG):
                chunk_copy(ids_v, c).wait()

    return k(g, embedding)


def kernel(ts, embedding):
    batch = ts.shape[0]
    g = _ranks_tc(ts)
    g1d = g.reshape(batch * SEQ)
    out = _scatter_sc(g1d, embedding, batch)
    return out.reshape(batch, SEQ, EXP)
